# baseline (speedup 1.0000x reference)
import functools

import jax
import jax.numpy as jnp
from jax import lax
from jax.experimental import pallas as pl
from jax.experimental.pallas import tpu as pltpu
from jax.experimental.pallas import tpu_sc as plsc

B = 4
N = 10000
F = 128
E = 320000
NC = 2
NS = 16
C = 80
EPS = E // NS
NCHUNK = EPS // C
EPP = NCHUNK * C
BN = N * B
N2 = N // NC
DUMP = N2
NDUMP = 504
ACC_ROWS = N2 + NDUMP
ZB_SUBS = 2
ZB_ROWS = ACC_ROWS // ZB_SUBS
WB_SUBS = 5
WB_ROWS = N2 // WB_SUBS


def _sc_deg(dst_local, ones128, zfeat):
    @functools.partial(
        pl.kernel,
        mesh=plsc.VectorSubcoreMesh(core_axis_name="c", subcore_axis_name="s"),
        out_type=jax.ShapeDtypeStruct((N, F), jnp.float32),
        scratch_types=[
            pltpu.VMEM((NCHUNK, C), jnp.int32),
            pltpu.VMEM((C, F), jnp.float32),
            pltpu.VMEM_SHARED((ACC_ROWS, F), jnp.float32),
        ],
    )
    def k(dst_hbm, ones_hbm, z_hbm, deg_hbm, didx, onesv, dacc):
        c = lax.axis_index("c")
        s = lax.axis_index("s")
        pltpu.sync_copy(dst_hbm.at[c].at[s], didx)
        pltpu.sync_copy(ones_hbm, onesv)

        @pl.when(s < ZB_SUBS)
        def _zero():
            off = pl.multiple_of(s * ZB_ROWS, 8)
            pltpu.sync_copy(z_hbm.at[pl.ds(off, ZB_ROWS)],
                            dacc.at[pl.ds(off, ZB_ROWS)])

        plsc.subcore_barrier()

        def body(i, carry):
            pltpu.sync_copy(onesv, dacc.at[didx.at[i]], add=True)
            return carry

        lax.fori_loop(0, NCHUNK, body, 0)
        plsc.subcore_barrier()

        @pl.when(s < WB_SUBS)
        def _wb():
            off = pl.multiple_of(s * WB_ROWS, 8)
            pltpu.sync_copy(
                dacc.at[pl.ds(off, WB_ROWS)],
                deg_hbm.at[pl.ds(pl.multiple_of(c * N2, 8) + off, WB_ROWS)])

    return k(dst_local, ones128, zfeat)


def _sc_prop(s_flat, src_all, dst_local, zfeat):
    @functools.partial(
        pl.kernel,
        mesh=plsc.VectorSubcoreMesh(core_axis_name="c", subcore_axis_name="s"),
        out_type=jax.ShapeDtypeStruct((BN, F), jnp.float32),
        scratch_types=[
            pltpu.VMEM((NCHUNK, C), jnp.int32),
            pltpu.VMEM((NCHUNK, C), jnp.int32),
            pltpu.VMEM((C, F), jnp.float32),
            pltpu.VMEM((C, F), jnp.float32),
            pltpu.VMEM_SHARED((ACC_ROWS, F), jnp.float32),
            pltpu.SemaphoreType.DMA,
            pltpu.SemaphoreType.DMA,
        ],
    )
    def k(s_hbm, src_hbm, dst_hbm, z_hbm, r_hbm,
          didx, sidx, rb0, rb1, acc, g0, g1):
        c = lax.axis_index("c")
        s = lax.axis_index("s")
        pltpu.sync_copy(dst_hbm.at[c].at[s], didx)
        for b in range(B):
            pltpu.sync_copy(src_hbm.at[c].at[b].at[s], sidx)

            @pl.when(s < ZB_SUBS)
            def _zero():
                off = pl.multiple_of(s * ZB_ROWS, 8)
                pltpu.sync_copy(z_hbm.at[pl.ds(off, ZB_ROWS)],
                                acc.at[pl.ds(off, ZB_ROWS)])

            plsc.subcore_barrier()

            pltpu.async_copy(s_hbm.at[sidx.at[0]], rb0, g0)
            pltpu.async_copy(s_hbm.at[sidx.at[1]], rb1, g1)

            def body(i, carry):
                pltpu.make_async_copy(s_hbm.at[sidx.at[0]], rb0, g0).wait()
                pltpu.sync_copy(rb0, acc.at[didx.at[2 * i]], add=True)

                @pl.when(i < NCHUNK // 2 - 1)
                def _pf0():
                    pltpu.async_copy(s_hbm.at[sidx.at[2 * i + 2]], rb0, g0)

                pltpu.make_async_copy(s_hbm.at[sidx.at[1]], rb1, g1).wait()
                pltpu.sync_copy(rb1, acc.at[didx.at[2 * i + 1]], add=True)

                @pl.when(i < NCHUNK // 2 - 1)
                def _pf1():
                    pltpu.async_copy(s_hbm.at[sidx.at[2 * i + 3]], rb1, g1)

                return carry

            lax.fori_loop(0, NCHUNK // 2, body, 0)
            plsc.subcore_barrier()

            @pl.when(s < WB_SUBS)
            def _wb():
                off = pl.multiple_of(s * WB_ROWS, 8)
                pltpu.sync_copy(
                    acc.at[pl.ds(off, WB_ROWS)],
                    r_hbm.at[pl.ds(
                        pl.multiple_of(b * N + c * N2, 8) + off, WB_ROWS)])

            plsc.subcore_barrier()

    return k(s_flat, src_all, dst_local, zfeat)


_R = 400
_NBLK = BN // _R
_NBLK_REP = N // _R

_row_spec = pl.BlockSpec((_R, F), lambda m: (m, 0))
_dinv_spec = pl.BlockSpec((_R, F), lambda m: (m % _NBLK_REP, 0))
_w_spec = pl.BlockSpec((F, F), lambda m: (0, 0))
_b_spec = pl.BlockSpec((1, F), lambda m: (0, 0))
_BN_F = jax.ShapeDtypeStruct((BN, F), jnp.float32)


def _tc_dinv(deg):
    def body(deg_ref, out_ref):
        d0 = deg_ref[:, 0:1]
        dinv = jnp.where(d0 > 0, lax.rsqrt(d0), 0.0)
        out_ref[...] = jnp.broadcast_to(dinv, (_R, F))

    return pl.pallas_call(
        body,
        grid=(_NBLK_REP,),
        in_specs=[pl.BlockSpec((_R, F), lambda m: (m, 0))],
        out_specs=pl.BlockSpec((_R, F), lambda m: (m, 0)),
        out_shape=jax.ShapeDtypeStruct((N, F), jnp.float32),
    )(deg)


def _tc_t1(xf, dinv, w):
    def body(x_ref, d_ref, w_ref, acc_ref, s_ref):
        h = x_ref[...]
        acc_ref[...] = jnp.dot(h, w_ref[...],
                               preferred_element_type=jnp.float32)
        s_ref[...] = h * d_ref[...]

    return pl.pallas_call(
        body,
        grid=(_NBLK,),
        in_specs=[_row_spec, _dinv_spec, _w_spec],
        out_specs=[_row_spec, _row_spec],
        out_shape=[_BN_F, _BN_F],
    )(xf, dinv, w)


def _tc_t2(r, dinv, w, acc):
    def body(r_ref, d_ref, w_ref, acc_ref, accout_ref, s_ref):
        d = d_ref[...]
        h = r_ref[...] * d
        accout_ref[...] = acc_ref[...] + jnp.dot(
            h, w_ref[...], preferred_element_type=jnp.float32)
        s_ref[...] = h * d

    return pl.pallas_call(
        body,
        grid=(_NBLK,),
        in_specs=[_row_spec, _dinv_spec, _w_spec, _row_spec],
        out_specs=[_row_spec, _row_spec],
        out_shape=[_BN_F, _BN_F],
        input_output_aliases={3: 0},
    )(r, dinv, w, acc)


def _tc_t3a(r, dinv, w13, b1, w20, acc):
    def body(r_ref, d_ref, w13_ref, b_ref, w20_ref, acc_ref,
             acc2_ref, s_ref):
        d = d_ref[...]
        h = acc_ref[...] + jnp.dot(r_ref[...] * d, w13_ref[...],
                                   preferred_element_type=jnp.float32)
        h = jnp.maximum(h + b_ref[...], 0.0)
        acc2_ref[...] = jnp.dot(h, w20_ref[...],
                                preferred_element_type=jnp.float32)
        s_ref[...] = h * d

    return pl.pallas_call(
        body,
        grid=(_NBLK,),
        in_specs=[_row_spec, _dinv_spec, _w_spec, _b_spec, _w_spec,
                  _row_spec],
        out_specs=[_row_spec, _row_spec],
        out_shape=[_BN_F, _BN_F],
        input_output_aliases={5: 0},
    )(r, dinv, w13, b1, w20, acc)


def _tc_t3b(r, dinv, w23, b2, acc):
    def body(r_ref, d_ref, w_ref, b_ref, acc_ref, out_ref):
        out_ref[...] = (acc_ref[...]
                        + jnp.dot(r_ref[...] * d_ref[...], w_ref[...],
                                  preferred_element_type=jnp.float32)
                        + b_ref[...])

    return pl.pallas_call(
        body,
        grid=(_NBLK,),
        in_specs=[_row_spec, _dinv_spec, _w_spec, _b_spec, _row_spec],
        out_specs=pl.BlockSpec((_R, F), lambda m: (m, 0)),
        out_shape=_BN_F,
        input_output_aliases={4: 0},
    )(r, dinv, w23, b2, acc)


def kernel(x, edge_index, W1, b1, W2, b2):
    src = edge_index[0]
    dst = edge_index[1]
    offs = (jnp.arange(B, dtype=jnp.int32) * N)[:, None]
    src_b = (src[None, :] + offs).reshape(B, NS, NCHUNK, C)
    src_all = jnp.stack([src_b, src_b])
    dump_idx = DUMP + (jnp.arange(E, dtype=jnp.int32) % NDUMP)
    dl0 = jnp.where(dst < N2, dst, dump_idx)
    dl1 = jnp.where(dst >= N2, dst - N2, dump_idx)
    dst_local = jnp.stack([dl0, dl1]).reshape(NC, NS, NCHUNK, C)
    zfeat = jnp.zeros((ACC_ROWS, F), jnp.float32)
    ones128 = jnp.ones((C, F), jnp.float32)

    deg = _sc_deg(dst_local, ones128, zfeat)
    dinv = _tc_dinv(deg)

    xf = x.reshape(BN, F)
    acc, s = _tc_t1(xf, dinv, W1[0])
    for k in (1, 2):
        r = _sc_prop(s, src_all, dst_local, zfeat)
        acc, s = _tc_t2(r, dinv, W1[k], acc)
    r = _sc_prop(s, src_all, dst_local, zfeat)
    acc2, s = _tc_t3a(r, dinv, W1[3], b1.reshape(1, F), W2[0], acc)
    for k in (1, 2):
        r = _sc_prop(s, src_all, dst_local, zfeat)
        acc2, s = _tc_t2(r, dinv, W2[k], acc2)
    r = _sc_prop(s, src_all, dst_local, zfeat)
    out = _tc_t3b(r, dinv, W2[3], b2.reshape(1, F), acc2)
    return out.reshape(B, N, F)


# speedup vs baseline: 1.0017x; 1.0017x over previous
import functools

import jax
import jax.numpy as jnp
from jax import lax
from jax.experimental import pallas as pl
from jax.experimental.pallas import tpu as pltpu
from jax.experimental.pallas import tpu_sc as plsc

B = 4
N = 10000
F = 128
E = 320000
NC = 2
NS = 16
C = 80
EPS = E // NS
NCHUNK = EPS // C
EPP = NCHUNK * C
BN = N * B
N2 = N // NC
DUMP = N2
NDUMP = 88
ACC_ROWS = N2 + NDUMP
ZB_SUBS = 2
ZB_ROWS = ACC_ROWS // ZB_SUBS
WB_SUBS = 5
WB_ROWS = N2 // WB_SUBS


def _sc_deg(dst_local, ones128, zfeat):
    @functools.partial(
        pl.kernel,
        mesh=plsc.VectorSubcoreMesh(core_axis_name="c", subcore_axis_name="s"),
        out_type=jax.ShapeDtypeStruct((N, F), jnp.float32),
        scratch_types=[
            pltpu.VMEM((NCHUNK, C), jnp.int32),
            pltpu.VMEM((C, F), jnp.float32),
            pltpu.VMEM_SHARED((ACC_ROWS, F), jnp.float32),
        ],
    )
    def k(dst_hbm, ones_hbm, z_hbm, deg_hbm, didx, onesv, dacc):
        c = lax.axis_index("c")
        s = lax.axis_index("s")
        pltpu.sync_copy(dst_hbm.at[c].at[s], didx)
        pltpu.sync_copy(ones_hbm, onesv)

        @pl.when(s < ZB_SUBS)
        def _zero():
            off = pl.multiple_of(s * ZB_ROWS, 8)
            pltpu.sync_copy(z_hbm.at[pl.ds(off, ZB_ROWS)],
                            dacc.at[pl.ds(off, ZB_ROWS)])

        plsc.subcore_barrier()

        def body(i, carry):
            pltpu.sync_copy(onesv, dacc.at[didx.at[i]], add=True)
            return carry

        lax.fori_loop(0, NCHUNK, body, 0)
        plsc.subcore_barrier()

        @pl.when(s < WB_SUBS)
        def _wb():
            off = pl.multiple_of(s * WB_ROWS, 8)
            pltpu.sync_copy(
                dacc.at[pl.ds(off, WB_ROWS)],
                deg_hbm.at[pl.ds(pl.multiple_of(c * N2, 8) + off, WB_ROWS)])

    return k(dst_local, ones128, zfeat)


def _sc_prop(s_flat, src_all, dst_local, zfeat):
    @functools.partial(
        pl.kernel,
        mesh=plsc.VectorSubcoreMesh(core_axis_name="c", subcore_axis_name="s"),
        out_type=jax.ShapeDtypeStruct((BN, F), jnp.float32),
        scratch_types=[
            pltpu.VMEM((NCHUNK, C), jnp.int32),
            pltpu.VMEM((NCHUNK, C), jnp.int32),
            pltpu.VMEM((C, F), jnp.float32),
            pltpu.VMEM((C, F), jnp.float32),
            pltpu.VMEM_SHARED((ACC_ROWS, F), jnp.float32),
            pltpu.SemaphoreType.DMA,
            pltpu.SemaphoreType.DMA,
        ],
    )
    def k(s_hbm, src_hbm, dst_hbm, z_hbm, r_hbm,
          didx, sidx, rb0, rb1, acc, g0, g1):
        c = lax.axis_index("c")
        s = lax.axis_index("s")
        pltpu.sync_copy(dst_hbm.at[c].at[s], didx)
        for b in range(B):
            pltpu.sync_copy(src_hbm.at[c].at[b].at[s], sidx)

            @pl.when(s < ZB_SUBS)
            def _zero():
                off = pl.multiple_of(s * ZB_ROWS, 8)
                pltpu.sync_copy(z_hbm.at[pl.ds(off, ZB_ROWS)],
                                acc.at[pl.ds(off, ZB_ROWS)])

            plsc.subcore_barrier()

            pltpu.async_copy(s_hbm.at[sidx.at[0]], rb0, g0)
            pltpu.async_copy(s_hbm.at[sidx.at[1]], rb1, g1)

            def body(i, carry):
                pltpu.make_async_copy(s_hbm.at[sidx.at[0]], rb0, g0).wait()
                pltpu.sync_copy(rb0, acc.at[didx.at[2 * i]], add=True)

                @pl.when(i < NCHUNK // 2 - 1)
                def _pf0():
                    pltpu.async_copy(s_hbm.at[sidx.at[2 * i + 2]], rb0, g0)

                pltpu.make_async_copy(s_hbm.at[sidx.at[1]], rb1, g1).wait()
                pltpu.sync_copy(rb1, acc.at[didx.at[2 * i + 1]], add=True)

                @pl.when(i < NCHUNK // 2 - 1)
                def _pf1():
                    pltpu.async_copy(s_hbm.at[sidx.at[2 * i + 3]], rb1, g1)

                return carry

            lax.fori_loop(0, NCHUNK // 2, body, 0)
            plsc.subcore_barrier()

            @pl.when(s < WB_SUBS)
            def _wb():
                off = pl.multiple_of(s * WB_ROWS, 8)
                pltpu.sync_copy(
                    acc.at[pl.ds(off, WB_ROWS)],
                    r_hbm.at[pl.ds(
                        pl.multiple_of(b * N + c * N2, 8) + off, WB_ROWS)])

            plsc.subcore_barrier()

    return k(s_flat, src_all, dst_local, zfeat)


_R = 400
_NBLK = BN // _R
_NBLK_REP = N // _R

_row_spec = pl.BlockSpec((_R, F), lambda m: (m, 0))
_dinv_spec = pl.BlockSpec((_R, F), lambda m: (m % _NBLK_REP, 0))
_w_spec = pl.BlockSpec((F, F), lambda m: (0, 0))
_b_spec = pl.BlockSpec((1, F), lambda m: (0, 0))
_BN_F = jax.ShapeDtypeStruct((BN, F), jnp.float32)


def _tc_dinv(deg):
    def body(deg_ref, out_ref):
        d0 = deg_ref[:, 0:1]
        dinv = jnp.where(d0 > 0, lax.rsqrt(d0), 0.0)
        out_ref[...] = jnp.broadcast_to(dinv, (_R, F))

    return pl.pallas_call(
        body,
        grid=(_NBLK_REP,),
        in_specs=[pl.BlockSpec((_R, F), lambda m: (m, 0))],
        out_specs=pl.BlockSpec((_R, F), lambda m: (m, 0)),
        out_shape=jax.ShapeDtypeStruct((N, F), jnp.float32),
    )(deg)


def _tc_t1(xf, dinv, w):
    def body(x_ref, d_ref, w_ref, acc_ref, s_ref):
        h = x_ref[...]
        acc_ref[...] = jnp.dot(h, w_ref[...],
                               preferred_element_type=jnp.float32)
        s_ref[...] = h * d_ref[...]

    return pl.pallas_call(
        body,
        grid=(_NBLK,),
        in_specs=[_row_spec, _dinv_spec, _w_spec],
        out_specs=[_row_spec, _row_spec],
        out_shape=[_BN_F, _BN_F],
    )(xf, dinv, w)


def _tc_t2(r, dinv, w, acc):
    def body(r_ref, d_ref, w_ref, acc_ref, accout_ref, s_ref):
        d = d_ref[...]
        h = r_ref[...] * d
        accout_ref[...] = acc_ref[...] + jnp.dot(
            h, w_ref[...], preferred_element_type=jnp.float32)
        s_ref[...] = h * d

    return pl.pallas_call(
        body,
        grid=(_NBLK,),
        in_specs=[_row_spec, _dinv_spec, _w_spec, _row_spec],
        out_specs=[_row_spec, _row_spec],
        out_shape=[_BN_F, _BN_F],
        input_output_aliases={3: 0},
    )(r, dinv, w, acc)


def _tc_t3a(r, dinv, w13, b1, w20, acc):
    def body(r_ref, d_ref, w13_ref, b_ref, w20_ref, acc_ref,
             acc2_ref, s_ref):
        d = d_ref[...]
        h = acc_ref[...] + jnp.dot(r_ref[...] * d, w13_ref[...],
                                   preferred_element_type=jnp.float32)
        h = jnp.maximum(h + b_ref[...], 0.0)
        acc2_ref[...] = jnp.dot(h, w20_ref[...],
                                preferred_element_type=jnp.float32)
        s_ref[...] = h * d

    return pl.pallas_call(
        body,
        grid=(_NBLK,),
        in_specs=[_row_spec, _dinv_spec, _w_spec, _b_spec, _w_spec,
                  _row_spec],
        out_specs=[_row_spec, _row_spec],
        out_shape=[_BN_F, _BN_F],
        input_output_aliases={5: 0},
    )(r, dinv, w13, b1, w20, acc)


def _tc_t3b(r, dinv, w23, b2, acc):
    def body(r_ref, d_ref, w_ref, b_ref, acc_ref, out_ref):
        out_ref[...] = (acc_ref[...]
                        + jnp.dot(r_ref[...] * d_ref[...], w_ref[...],
                                  preferred_element_type=jnp.float32)
                        + b_ref[...])

    return pl.pallas_call(
        body,
        grid=(_NBLK,),
        in_specs=[_row_spec, _dinv_spec, _w_spec, _b_spec, _row_spec],
        out_specs=pl.BlockSpec((_R, F), lambda m: (m, 0)),
        out_shape=_BN_F,
        input_output_aliases={4: 0},
    )(r, dinv, w23, b2, acc)


def kernel(x, edge_index, W1, b1, W2, b2):
    src = edge_index[0]
    dst = edge_index[1]
    offs = (jnp.arange(B, dtype=jnp.int32) * N)[:, None]
    src_b = (src[None, :] + offs).reshape(B, NS, NCHUNK, C)
    src_all = jnp.stack([src_b, src_b])
    dump_idx = DUMP + (jnp.arange(E, dtype=jnp.int32) % NDUMP)
    dl0 = jnp.where(dst < N2, dst, dump_idx)
    dl1 = jnp.where(dst >= N2, dst - N2, dump_idx)
    dst_local = jnp.stack([dl0, dl1]).reshape(NC, NS, NCHUNK, C)
    zfeat = jnp.zeros((ACC_ROWS, F), jnp.float32)
    ones128 = jnp.ones((C, F), jnp.float32)

    deg = _sc_deg(dst_local, ones128, zfeat)
    dinv = _tc_dinv(deg)

    xf = x.reshape(BN, F)
    acc, s = _tc_t1(xf, dinv, W1[0])
    for k in (1, 2):
        r = _sc_prop(s, src_all, dst_local, zfeat)
        acc, s = _tc_t2(r, dinv, W1[k], acc)
    r = _sc_prop(s, src_all, dst_local, zfeat)
    acc2, s = _tc_t3a(r, dinv, W1[3], b1.reshape(1, F), W2[0], acc)
    for k in (1, 2):
        r = _sc_prop(s, src_all, dst_local, zfeat)
        acc2, s = _tc_t2(r, dinv, W2[k], acc2)
    r = _sc_prop(s, src_all, dst_local, zfeat)
    out = _tc_t3b(r, dinv, W2[3], b2.reshape(1, F), acc2)
    return out.reshape(B, N, F)
